# baseline (speedup 1.0000x reference)
import functools

import jax
import jax.numpy as jnp
from jax import lax
from jax.experimental import pallas as pl
from jax.experimental.pallas import tpu as pltpu
from jax.experimental.pallas import tpu_sc as plsc

N_NODES = 10000
N_PAD = 10240
F = 128
N_EDGES = 320000
NC, NS = 2, 16
CHUNK = 128
NCHUNK = 80
NCHUNK_TOT = NC * NS * NCHUNK
C_SLOW = 48
C_FAST = NCHUNK_TOT // NS - C_SLOW
MAXC = max(C_SLOW, C_FAST)
SLOW_CID = 1
E_PAD = (NCHUNK_TOT + MAXC) * CHUNK
ROWS_PER_TILE = N_PAD // NS
DUMMY_DST = 10200

_sc_mesh = plsc.VectorSubcoreMesh(core_axis_name="c", subcore_axis_name="s")


@functools.partial(
    pl.kernel,
    out_type=jax.ShapeDtypeStruct((NC * NS, N_PAD), jnp.float32),
    mesh=_sc_mesh,
    compiler_params=pltpu.CompilerParams(needs_layout_passes=False),
    scratch_types=[
        pltpu.VMEM((NCHUNK, CHUNK), jnp.int32),
        pltpu.VMEM((N_PAD,), jnp.float32),
    ],
)
def _deg_kernel(dst_hbm, zeros_hbm, out_hbm, dst_v, deg_v):
    cid = lax.axis_index("c")
    sid = lax.axis_index("s")
    wid = sid * NC + cid
    pltpu.sync_copy(zeros_hbm, deg_v)
    pltpu.sync_copy(dst_hbm.at[pl.ds(wid * NCHUNK, NCHUNK)], dst_v)
    ones = jnp.ones((16,), jnp.float32)

    def body(j, carry):
        for k in range(CHUNK // 16):
            idx = dst_v[j, pl.ds(k * 16, 16)]
            plsc.addupdate_scatter(deg_v, [idx], ones)
        return carry

    lax.fori_loop(0, NCHUNK, body, 0)
    pltpu.sync_copy(deg_v, out_hbm.at[wid])


@functools.partial(
    pl.kernel,
    out_type=jax.ShapeDtypeStruct((NC, N_PAD, F), jnp.float32),
    mesh=_sc_mesh,
    scratch_types=[
        pltpu.VMEM_SHARED((N_PAD, F), jnp.float32),
        pltpu.VMEM((MAXC, CHUNK), jnp.int32),
        pltpu.VMEM((MAXC, CHUNK), jnp.int32),
        pltpu.VMEM((CHUNK, F), jnp.float32),
        pltpu.SemaphoreType.DMA,
    ],
)
def _scatter_kernel(xs_hbm, src_hbm, dst_hbm, out_hbm,
                    acc_sh, src_v, dst_v, rows_v, sem):
    cid = lax.axis_index("c")
    sid = lax.axis_index("s")
    wid = sid * NC + cid
    base = sid * ROWS_PER_TILE

    def zbody(i, c):
        for k in range(F // 16):
            rows_v[i, pl.ds(k * 16, 16)] = jnp.zeros((16,), jnp.float32)
        return c
    lax.fori_loop(0, CHUNK, zbody, 0)
    for m in range(ROWS_PER_TILE // CHUNK):
        pltpu.sync_copy(rows_v, acc_sh.at[pl.ds(base + m * CHUNK, CHUNK)])

    plsc.subcore_barrier()

    start = jnp.where(cid == SLOW_CID, sid * C_SLOW,
                      NS * C_SLOW + sid * C_FAST)
    nch = jnp.where(cid == SLOW_CID, C_SLOW, C_FAST)
    pltpu.sync_copy(src_hbm.at[pl.ds(start, MAXC)], src_v)
    pltpu.sync_copy(dst_hbm.at[pl.ds(start, MAXC)], dst_v)

    def body(j, carry):
        pltpu.async_copy(xs_hbm.at[src_v.at[j]], rows_v, sem).wait()
        pltpu.sync_copy(rows_v, acc_sh.at[dst_v.at[j]], add=True)
        return carry

    lax.fori_loop(0, nch, body, 0)

    plsc.subcore_barrier()
    pltpu.sync_copy(acc_sh.at[pl.ds(base, ROWS_PER_TILE)],
                    out_hbm.at[cid, pl.ds(base, ROWS_PER_TILE)])


BM = 512
GRID = N_PAD // BM


def _dinv(d_ref):
    deg = jnp.sum(d_ref[...], axis=0) + 1.0
    return lax.rsqrt(deg)[:, None]


def _k1_body(x_ref, w_ref, d_ref, o_ref):
    mm = jnp.dot(x_ref[...], w_ref[...], preferred_element_type=jnp.float32, precision=lax.Precision.HIGHEST)
    o_ref[...] = _dinv(d_ref) * mm


def _k2_body(p0_ref, p1_ref, xs_ref, d_ref, b_ref, w_ref, o_ref):
    dinv = _dinv(d_ref)
    h = jnp.maximum(
        dinv * (p0_ref[...] + p1_ref[...] + xs_ref[...]) + b_ref[...], 0.0)
    o_ref[...] = dinv * jnp.dot(h, w_ref[...], preferred_element_type=jnp.float32, precision=lax.Precision.HIGHEST)


def _k3_body(q0_ref, q1_ref, xs_ref, d_ref, b_ref, o_ref):
    dinv = _dinv(d_ref)
    o_ref[...] = dinv * (q0_ref[...] + q1_ref[...] + xs_ref[...]) + b_ref[...]


_row_spec = pl.BlockSpec((BM, F), lambda i: (i, 0))
_deg_spec = pl.BlockSpec((NC * NS, BM), lambda i: (0, i))
_full_spec = pl.BlockSpec((F, F), lambda i: (0, 0))
_b_spec = pl.BlockSpec((1, F), lambda i: (0, 0))
_out_t = jax.ShapeDtypeStruct((N_PAD, F), jnp.float32)

_k1 = pl.pallas_call(
    _k1_body, grid=(GRID,),
    in_specs=[_row_spec, _full_spec, _deg_spec],
    out_specs=_row_spec, out_shape=_out_t)

_k2 = pl.pallas_call(
    _k2_body, grid=(GRID,),
    in_specs=[_row_spec, _row_spec, _row_spec, _deg_spec, _b_spec, _full_spec],
    out_specs=_row_spec, out_shape=_out_t)

_k3 = pl.pallas_call(
    _k3_body, grid=(GRID,),
    in_specs=[_row_spec, _row_spec, _row_spec, _deg_spec, _b_spec],
    out_specs=_row_spec, out_shape=_out_t)


def kernel(edge_index, x, W1, b1, W2, b2):
    src = edge_index[0].astype(jnp.int32)
    dst = edge_index[1].astype(jnp.int32)
    pad = E_PAD - N_EDGES
    pad_dst = N_NODES + jnp.arange(pad, dtype=jnp.int32) % (N_PAD - N_NODES)
    src_p = jnp.concatenate(
        [src, jnp.full((pad,), N_NODES, jnp.int32)]).reshape(NCHUNK_TOT + MAXC, CHUNK)
    dst_p = jnp.concatenate([dst, pad_dst]).reshape(NCHUNK_TOT + MAXC, CHUNK)
    x_pad = jnp.zeros((N_PAD, F), jnp.float32).at[:N_NODES].set(x)
    zeros_n = jnp.zeros((N_PAD,), jnp.float32)
    b1r = b1.reshape(1, F)
    b2r = b2.reshape(1, F)

    deg = _deg_kernel(dst_p, zeros_n)

    xs1 = _k1(x_pad, W1, deg)
    acc1 = _scatter_kernel(xs1, src_p, dst_p)
    xs2 = _k2(acc1[0], acc1[1], xs1, deg, b1r, W2)
    acc2 = _scatter_kernel(xs2, src_p, dst_p)
    out = _k3(acc2[0], acc2[1], xs2, deg, b2r)
    return out[:N_NODES]


# speedup vs baseline: 2.6217x; 2.6217x over previous
import functools

import jax
import jax.numpy as jnp
from jax import lax
from jax.experimental import pallas as pl
from jax.experimental.pallas import tpu as pltpu
from jax.experimental.pallas import tpu_sc as plsc

N_NODES = 10000
N_PAD = 10240
F = 128
N_EDGES = 320000
NC, NS = 2, 16
CHUNK = 128
NCHUNK = 80
NCHUNK_TOT = NC * NS * NCHUNK
C_SLOW = 80
C_FAST = NCHUNK_TOT // NS - C_SLOW
MAXC = max(C_SLOW, C_FAST)
SLOW_CID = 1
E_PAD = (NCHUNK_TOT + MAXC) * CHUNK
ROWS_PER_TILE = N_PAD // NS
DUMMY_DST = 10200

_sc_mesh = plsc.VectorSubcoreMesh(core_axis_name="c", subcore_axis_name="s")


@functools.partial(
    pl.kernel,
    out_type=jax.ShapeDtypeStruct((NC * NS, N_PAD), jnp.float32),
    mesh=_sc_mesh,
    compiler_params=pltpu.CompilerParams(needs_layout_passes=False),
    scratch_types=[
        pltpu.VMEM((NCHUNK, CHUNK), jnp.int32),
        pltpu.VMEM((N_PAD,), jnp.float32),
    ],
)
def _deg_kernel(dst_hbm, zeros_hbm, out_hbm, dst_v, deg_v):
    cid = lax.axis_index("c")
    sid = lax.axis_index("s")
    wid = sid * NC + cid
    pltpu.sync_copy(zeros_hbm, deg_v)
    pltpu.sync_copy(dst_hbm.at[pl.ds(wid * NCHUNK, NCHUNK)], dst_v)
    ones = jnp.ones((16,), jnp.float32)

    def body(j, carry):
        for k in range(CHUNK // 16):
            idx = dst_v[j, pl.ds(k * 16, 16)]
            plsc.addupdate_scatter(deg_v, [idx], ones)
        return carry

    lax.fori_loop(0, NCHUNK, body, 0)
    pltpu.sync_copy(deg_v, out_hbm.at[wid])


@functools.partial(
    pl.kernel,
    out_type=jax.ShapeDtypeStruct((NC, N_PAD, F), jnp.float32),
    mesh=_sc_mesh,
    scratch_types=[
        pltpu.VMEM_SHARED((N_PAD, F), jnp.float32),
        pltpu.VMEM((MAXC, CHUNK), jnp.int32),
        pltpu.VMEM((MAXC, CHUNK), jnp.int32),
        pltpu.VMEM((CHUNK, F), jnp.float32),
        pltpu.SemaphoreType.DMA,
    ],
)
def _scatter_kernel(xs_hbm, src_hbm, dst_hbm, out_hbm,
                    acc_sh, src_v, dst_v, rows_v, sem):
    cid = lax.axis_index("c")
    sid = lax.axis_index("s")
    wid = sid * NC + cid
    base = sid * ROWS_PER_TILE

    def zbody(i, c):
        for k in range(F // 16):
            rows_v[i, pl.ds(k * 16, 16)] = jnp.zeros((16,), jnp.float32)
        return c
    lax.fori_loop(0, CHUNK, zbody, 0)
    for m in range(ROWS_PER_TILE // CHUNK):
        pltpu.sync_copy(rows_v, acc_sh.at[pl.ds(base + m * CHUNK, CHUNK)])

    plsc.subcore_barrier()

    start = jnp.where(cid == SLOW_CID, sid * C_SLOW,
                      NS * C_SLOW + sid * C_FAST)
    nch = jnp.where(cid == SLOW_CID, C_SLOW, C_FAST)
    pltpu.sync_copy(src_hbm.at[pl.ds(start, MAXC)], src_v)
    pltpu.sync_copy(dst_hbm.at[pl.ds(start, MAXC)], dst_v)

    def body(j, carry):
        pltpu.async_copy(xs_hbm.at[src_v.at[j]], rows_v, sem).wait()
        pltpu.sync_copy(rows_v, acc_sh.at[dst_v.at[j]], add=True)
        return carry

    lax.fori_loop(0, nch, body, 0)

    plsc.subcore_barrier()
    pltpu.sync_copy(acc_sh.at[pl.ds(base, ROWS_PER_TILE)],
                    out_hbm.at[cid, pl.ds(base, ROWS_PER_TILE)])


BM = 512
GRID = N_PAD // BM


def _dinv(d_ref):
    deg = jnp.sum(d_ref[...], axis=0) + 1.0
    return lax.rsqrt(deg)[:, None]


def _k1_body(x_ref, w_ref, d_ref, o_ref):
    mm = jnp.dot(x_ref[...], w_ref[...], preferred_element_type=jnp.float32, precision=lax.Precision.HIGHEST)
    o_ref[...] = _dinv(d_ref) * mm


def _k2_body(p0_ref, p1_ref, xs_ref, d_ref, b_ref, w_ref, o_ref):
    dinv = _dinv(d_ref)
    h = jnp.maximum(
        dinv * (p0_ref[...] + p1_ref[...] + xs_ref[...]) + b_ref[...], 0.0)
    o_ref[...] = dinv * jnp.dot(h, w_ref[...], preferred_element_type=jnp.float32, precision=lax.Precision.HIGHEST)


def _k3_body(q0_ref, q1_ref, xs_ref, d_ref, b_ref, o_ref):
    dinv = _dinv(d_ref)
    o_ref[...] = dinv * (q0_ref[...] + q1_ref[...] + xs_ref[...]) + b_ref[...]


_row_spec = pl.BlockSpec((BM, F), lambda i: (i, 0))
_deg_spec = pl.BlockSpec((NC * NS, BM), lambda i: (0, i))
_full_spec = pl.BlockSpec((F, F), lambda i: (0, 0))
_b_spec = pl.BlockSpec((1, F), lambda i: (0, 0))
_out_t = jax.ShapeDtypeStruct((N_PAD, F), jnp.float32)

_k1 = pl.pallas_call(
    _k1_body, grid=(GRID,),
    in_specs=[_row_spec, _full_spec, _deg_spec],
    out_specs=_row_spec, out_shape=_out_t)

_k2 = pl.pallas_call(
    _k2_body, grid=(GRID,),
    in_specs=[_row_spec, _row_spec, _row_spec, _deg_spec, _b_spec, _full_spec],
    out_specs=_row_spec, out_shape=_out_t)

_k3 = pl.pallas_call(
    _k3_body, grid=(GRID,),
    in_specs=[_row_spec, _row_spec, _row_spec, _deg_spec, _b_spec],
    out_specs=_row_spec, out_shape=_out_t)


def kernel(edge_index, x, W1, b1, W2, b2):
    src = edge_index[0].astype(jnp.int32)
    dst = edge_index[1].astype(jnp.int32)
    pad = E_PAD - N_EDGES
    pad_dst = N_NODES + jnp.arange(pad, dtype=jnp.int32) % (N_PAD - N_NODES)
    pad_src = jnp.arange(pad, dtype=jnp.int32) % N_PAD
    src_p = jnp.concatenate([src, pad_src]).reshape(NCHUNK_TOT + MAXC, CHUNK)
    dst_p = jnp.concatenate([dst, pad_dst]).reshape(NCHUNK_TOT + MAXC, CHUNK)
    x_pad = jnp.zeros((N_PAD, F), jnp.float32).at[:N_NODES].set(x)
    zeros_n = jnp.zeros((N_PAD,), jnp.float32)
    b1r = b1.reshape(1, F)
    b2r = b2.reshape(1, F)

    deg = _deg_kernel(dst_p, zeros_n)

    xs1 = _k1(x_pad, W1, deg)
    acc1 = _scatter_kernel(xs1, src_p, dst_p)
    xs2 = _k2(acc1[0], acc1[1], xs1, deg, b1r, W2)
    acc2 = _scatter_kernel(xs2, src_p, dst_p)
    out = _k3(acc2[0], acc2[1], xs2, deg, b2r)
    return out[:N_NODES]


# speedup vs baseline: 2.6224x; 1.0003x over previous
import functools

import jax
import jax.numpy as jnp
from jax import lax
from jax.experimental import pallas as pl
from jax.experimental.pallas import tpu as pltpu
from jax.experimental.pallas import tpu_sc as plsc

N_NODES = 10000
N_PAD = 10240
F = 128
N_EDGES = 320000
NC, NS = 2, 16
CHUNK = 128
NCHUNK = 80
NCHUNK_TOT = NC * NS * NCHUNK
C_SLOW = 80
C_FAST = NCHUNK_TOT // NS - C_SLOW
MAXC = max(C_SLOW, C_FAST)
SLOW_CID = 1
E_PAD = (NCHUNK_TOT + MAXC) * CHUNK
ROWS_PER_TILE = N_PAD // NS

_sc_mesh = plsc.VectorSubcoreMesh(core_axis_name="c", subcore_axis_name="s")


@functools.partial(
    pl.kernel,
    out_type=jax.ShapeDtypeStruct((NC * NS, N_PAD), jnp.float32),
    mesh=_sc_mesh,
    compiler_params=pltpu.CompilerParams(needs_layout_passes=False),
    scratch_types=[
        pltpu.VMEM((NCHUNK, CHUNK), jnp.int32),
        pltpu.VMEM((N_PAD,), jnp.float32),
    ],
)
def _deg_kernel(dst_hbm, zeros_hbm, out_hbm, dst_v, deg_v):
    cid = lax.axis_index("c")
    sid = lax.axis_index("s")
    wid = sid * NC + cid
    pltpu.sync_copy(zeros_hbm, deg_v)
    pltpu.sync_copy(dst_hbm.at[pl.ds(wid * NCHUNK, NCHUNK)], dst_v)
    ones = jnp.ones((16,), jnp.float32)

    def body(j, carry):
        for k in range(CHUNK // 16):
            idx = dst_v[j, pl.ds(k * 16, 16)]
            plsc.addupdate_scatter(deg_v, [idx], ones)
        return carry

    lax.fori_loop(0, NCHUNK, body, 0)
    pltpu.sync_copy(deg_v, out_hbm.at[wid])


@functools.partial(
    pl.kernel,
    out_type=jax.ShapeDtypeStruct((NC, N_PAD, F), jnp.float32),
    mesh=_sc_mesh,
    scratch_types=[
        pltpu.VMEM_SHARED((N_PAD, F), jnp.float32),
        pltpu.VMEM((MAXC, CHUNK), jnp.int32),
        pltpu.VMEM((MAXC, CHUNK), jnp.int32),
        pltpu.VMEM((CHUNK, F), jnp.float32),
        pltpu.SemaphoreType.DMA,
    ],
)
def _scatter_kernel(xs_hbm, src_hbm, dst_hbm, out_hbm,
                    acc_sh, src_v, dst_v, rows_v, sem):
    cid = lax.axis_index("c")
    sid = lax.axis_index("s")
    wid = sid * NC + cid
    base = sid * ROWS_PER_TILE

    def zbody(i, c):
        for k in range(F // 16):
            rows_v[i, pl.ds(k * 16, 16)] = jnp.zeros((16,), jnp.float32)
        return c
    lax.fori_loop(0, CHUNK, zbody, 0)
    for m in range(ROWS_PER_TILE // CHUNK):
        pltpu.sync_copy(rows_v, acc_sh.at[pl.ds(base + m * CHUNK, CHUNK)])

    plsc.subcore_barrier()

    start = jnp.where(cid == SLOW_CID, sid * C_SLOW,
                      NS * C_SLOW + sid * C_FAST)
    nch = jnp.where(cid == SLOW_CID, C_SLOW, C_FAST)
    pltpu.sync_copy(src_hbm.at[pl.ds(start, MAXC)], src_v)
    pltpu.sync_copy(dst_hbm.at[pl.ds(start, MAXC)], dst_v)

    def body(j, carry):
        pltpu.async_copy(xs_hbm.at[src_v.at[j]], rows_v, sem).wait()
        pltpu.sync_copy(rows_v, acc_sh.at[dst_v.at[j]], add=True)
        return carry

    lax.fori_loop(0, nch, body, 0)

    plsc.subcore_barrier()
    pltpu.sync_copy(acc_sh.at[pl.ds(base, ROWS_PER_TILE)],
                    out_hbm.at[cid, pl.ds(base, ROWS_PER_TILE)])


BM = 512
GRID = N_PAD // BM


def _dinv(d_ref):
    deg = jnp.sum(d_ref[...], axis=0) + 1.0
    return lax.rsqrt(deg)[:, None]


def _k1_body(x_ref, w_ref, d_ref, o_ref):
    mm = jnp.dot(x_ref[...], w_ref[...], preferred_element_type=jnp.float32, precision=lax.Precision.HIGHEST)
    o_ref[...] = _dinv(d_ref) * mm


def _k2_body(p0_ref, p1_ref, xs_ref, d_ref, b_ref, w_ref, o_ref):
    dinv = _dinv(d_ref)
    h = jnp.maximum(
        dinv * (p0_ref[...] + p1_ref[...] + xs_ref[...]) + b_ref[...], 0.0)
    o_ref[...] = dinv * jnp.dot(h, w_ref[...], preferred_element_type=jnp.float32, precision=lax.Precision.HIGHEST)


def _k3_body(q0_ref, q1_ref, xs_ref, d_ref, b_ref, o_ref):
    dinv = _dinv(d_ref)
    o_ref[...] = dinv * (q0_ref[...] + q1_ref[...] + xs_ref[...]) + b_ref[...]


_row_spec = pl.BlockSpec((BM, F), lambda i: (i, 0))
_deg_spec = pl.BlockSpec((NC * NS, BM), lambda i: (0, i))
_full_spec = pl.BlockSpec((F, F), lambda i: (0, 0))
_b_spec = pl.BlockSpec((1, F), lambda i: (0, 0))
_out_t = jax.ShapeDtypeStruct((N_PAD, F), jnp.float32)

_k1 = pl.pallas_call(
    _k1_body, grid=(GRID,),
    in_specs=[_row_spec, _full_spec, _deg_spec],
    out_specs=_row_spec, out_shape=_out_t)

_k2 = pl.pallas_call(
    _k2_body, grid=(GRID,),
    in_specs=[_row_spec, _row_spec, _row_spec, _deg_spec, _b_spec, _full_spec],
    out_specs=_row_spec, out_shape=_out_t)

_k3 = pl.pallas_call(
    _k3_body, grid=(GRID,),
    in_specs=[_row_spec, _row_spec, _row_spec, _deg_spec, _b_spec],
    out_specs=_row_spec, out_shape=_out_t)


def kernel(edge_index, x, W1, b1, W2, b2):
    src = edge_index[0].astype(jnp.int32)
    dst = edge_index[1].astype(jnp.int32)
    pad = E_PAD - N_EDGES
    pad_dst = N_NODES + jnp.arange(pad, dtype=jnp.int32) % (N_PAD - N_NODES)
    pad_src = jnp.arange(pad, dtype=jnp.int32) % N_PAD
    src_p = jnp.concatenate([src, pad_src]).reshape(NCHUNK_TOT + MAXC, CHUNK)
    dst_p = jnp.concatenate([dst, pad_dst]).reshape(NCHUNK_TOT + MAXC, CHUNK)
    x_pad = jnp.zeros((N_PAD, F), jnp.float32).at[:N_NODES].set(x)
    zeros_n = jnp.zeros((N_PAD,), jnp.float32)
    b1r = b1.reshape(1, F)
    b2r = b2.reshape(1, F)

    deg = _deg_kernel(dst_p, zeros_n)

    xs1 = _k1(x_pad, W1, deg)
    acc1 = _scatter_kernel(xs1, src_p, dst_p)
    xs2 = _k2(acc1[0], acc1[1], xs1, deg, b1r, W2)
    acc2 = _scatter_kernel(xs2, src_p, dst_p)
    out = _k3(acc2[0], acc2[1], xs2, deg, b2r)
    return out[:N_NODES]
